# baseline (speedup 1.0000x reference)
import functools

import jax
import jax.numpy as jnp
from jax import lax
from jax.experimental import pallas as pl
from jax.experimental.pallas import tpu as pltpu
from jax.experimental.pallas import tpu_sc as plsc

B = 64
S = 2048
H = 64
NC = 2
NS = 16
NW = NC * NS
SPW = S // NW
LANES = 16


def _emb_body(xw_hbm, wt_hbm, pos_hbm, out_hbm, idx_v, pbuf, gbuf, sem):
  wid = lax.axis_index("s") * NC + lax.axis_index("c")
  s0 = wid * SPW

  pltpu.sync_copy(pos_hbm.at[pl.ds(s0, SPW)], pbuf)
  pltpu.sync_copy(xw_hbm.at[wid], idx_v)

  @pl.loop(0, B)
  def _chunk(b):
    pltpu.async_copy(wt_hbm.at[idx_v.at[b]], gbuf, sem).wait()
    @pl.loop(0, SPW)
    def _row(r):
      for h in range(H // LANES):
        sl = pl.ds(h * LANES, LANES)
        plsc.addupdate(gbuf.at[r, sl], pbuf[r, sl])
    pltpu.sync_copy(gbuf, out_hbm.at[pl.ds(b * S + s0, SPW)])


@jax.jit
def _emb(xw, word_table, pos_table):
  mesh = plsc.VectorSubcoreMesh(
      core_axis_name="c", subcore_axis_name="s", num_cores=NC, num_subcores=NS
  )
  return pl.kernel(
      _emb_body,
      out_type=jax.ShapeDtypeStruct((B * S, H), jnp.float32),
      mesh=mesh,
      scratch_types=[
          pltpu.VMEM((B, SPW), jnp.int32),
          pltpu.VMEM((SPW, H), jnp.float32),
          pltpu.VMEM((SPW, H), jnp.float32),
          pltpu.SemaphoreType.DMA,
      ],
      compiler_params=pltpu.CompilerParams(use_tc_tiling_on_sc=False),
  )(xw, word_table, pos_table)


def kernel(x, word_table, pos_table):
  x = x.astype(jnp.int32)
  xw = x.reshape(B, NW, SPW).transpose(1, 0, 2)
  out = _emb(xw, word_table, pos_table)
  return out.reshape(B, S, H)
